# Initial kernel scaffold; baseline (speedup 1.0000x reference)
import jax
import jax.numpy as jnp
from jax.experimental import pallas as pl


def kernel(h, params):
    raise NotImplementedError("write your pallas kernel here")


# speedup vs baseline: 8.9509x; 8.9509x over previous
import jax
import jax.numpy as jnp
from jax.experimental import pallas as pl
from jax.experimental.pallas import tpu as pltpu

NLAT = 32
NSEN = 512
HID = 256
INF = 128
OUTF = 128
SCH = 128
NCH = NSEN // SCH
NLAYERS = 4


def _silu(x):
    return x * jax.nn.sigmoid(x)


def _edge_block(U, V, We2, be2, WaT, ba):
    nu, H = U.shape
    nv = V.shape[0]
    E1 = _silu(U[:, None, :] + V[None, :, :])
    E1 = E1.reshape(nu * nv, H)
    E2 = _silu(jnp.dot(E1, We2, preferred_element_type=jnp.float32) + be2)
    att = jax.nn.sigmoid(jnp.sum(E2 * WaT, axis=1, keepdims=True) + ba)
    Eatt = E2 * att
    return Eatt.reshape(nu, nv, H).sum(axis=1)


def _node_update(hpart, agg, Wn1a, Wn1b, bn1, Wn2, bn2):
    m = _silu(jnp.dot(hpart, Wn1a, preferred_element_type=jnp.float32)
              + jnp.dot(agg, Wn1b, preferred_element_type=jnp.float32) + bn1)
    out = jnp.dot(m, Wn2, preferred_element_type=jnp.float32) + bn2
    return hpart + out


def _fwd_kernel(h_ref, idl_ref, Wlin_ref, blin_ref, Win_ref, bin_ref,
                Wout_ref, bout_ref, *rest):
    layer_refs, o_ref = rest[:-1], rest[-1]
    h = h_ref[0]
    lat = jnp.dot(idl_ref[...], Wlin_ref[...],
                  preferred_element_type=jnp.float32) + blin_ref[...]
    hl = jnp.dot(lat, Win_ref[...],
                 preferred_element_type=jnp.float32) + bin_ref[...]
    hs = jnp.dot(h, Win_ref[...],
                 preferred_element_type=jnp.float32) + bin_ref[...]

    for i in range(NLAYERS):
        (We1a, We1b, be1, We2, be2, WaT, ba,
         Wn1a, Wn1b, bn1, Wn2, bn2) = [r[...] for r in layer_refs[12 * i:12 * i + 12]]
        if i % 2 == 0:
            U = jnp.dot(hl, We1a, preferred_element_type=jnp.float32) + be1
            V = jnp.dot(hs, We1b, preferred_element_type=jnp.float32)
            agg = jnp.zeros((NLAT, HID), jnp.float32)
            for c in range(NCH):
                agg = agg + _edge_block(U, V[c * SCH:(c + 1) * SCH],
                                        We2, be2, WaT, ba)
            hl = _node_update(hl, agg, Wn1a, Wn1b, bn1, Wn2, bn2)
        else:
            U = jnp.dot(hs, We1a, preferred_element_type=jnp.float32) + be1
            V = jnp.dot(hl, We1b, preferred_element_type=jnp.float32)
            chunks = []
            for c in range(NCH):
                sl = slice(c * SCH, (c + 1) * SCH)
                agg_c = _edge_block(U[sl], V, We2, be2, WaT, ba) * (2.0 / NLAT)
                chunks.append(_node_update(hs[sl], agg_c,
                                           Wn1a, Wn1b, bn1, Wn2, bn2))
            hs = jnp.concatenate(chunks, axis=0)

    o_ref[0] = jnp.dot(hs, Wout_ref[...],
                       preferred_element_type=jnp.float32) + bout_ref[...]


def kernel(h, params):
    p = params
    row = lambda b: b.reshape(1, -1)

    args = [
        p["id_latent"].reshape(NLAT, -1),
        p["W_lin"], row(p["b_lin"]),
        p["W_in"], row(p["b_in"]),
        p["W_out"], row(p["b_out"]),
    ]
    for i in range(NLAYERS):
        g = p["gcl_%d" % i]
        args += [
            g["We1"][:HID], g["We1"][HID:], row(g["be1"]),
            g["We2"], row(g["be2"]),
            g["Wa"].reshape(1, HID), g["ba"].reshape(1, 1),
            g["Wn1"][:HID], g["Wn1"][HID:], row(g["bn1"]),
            g["Wn2"], row(g["bn2"]),
        ]

    full = lambda a: pl.BlockSpec(a.shape, lambda b: (0,) * a.ndim)
    out = pl.pallas_call(
        _fwd_kernel,
        grid=(h.shape[0],),
        in_specs=[pl.BlockSpec((1, NSEN, INF), lambda b: (b, 0, 0))]
                 + [full(a) for a in args],
        out_specs=pl.BlockSpec((1, NSEN, OUTF), lambda b: (b, 0, 0)),
        out_shape=jax.ShapeDtypeStruct((h.shape[0], NSEN, OUTF), jnp.float32),
        compiler_params=pltpu.CompilerParams(
            dimension_semantics=("parallel",)),
    )(h, *args)
    return out
